# baseline (speedup 1.0000x reference)
import functools

import jax
import jax.numpy as jnp
from jax import lax
from jax.experimental import pallas as pl
from jax.experimental.pallas import tpu as pltpu
from jax.experimental.pallas import tpu_sc as plsc

N_NODES = 10000
D_FEAT = 128
HALF = 64
N_EDGES = 320000

NUM_CORES = 2
NUM_SUBCORES = 16
NW = NUM_CORES * NUM_SUBCORES
EDGES_PER_W = N_EDGES // NW
CHUNK = 400
NCHUNKS = EDGES_PER_W // CHUNK
GROUPS = CHUNK // 16
NPAIRS = NCHUNKS // 2


def _make_kernel():
    mesh = plsc.VectorSubcoreMesh(core_axis_name="c", subcore_axis_name="s")

    @functools.partial(
        pl.kernel,
        mesh=mesh,
        compiler_params=pltpu.CompilerParams(
            needs_layout_passes=False, use_tc_tiling_on_sc=False),
        out_type=jax.ShapeDtypeStruct((N_EDGES,), jnp.float32),
        scratch_types=[
            pltpu.VMEM((CHUNK,), jnp.int32),
            pltpu.VMEM((CHUNK,), jnp.int32),
            pltpu.VMEM((CHUNK, HALF), jnp.float32),
            pltpu.VMEM((CHUNK, HALF), jnp.float32),
            pltpu.VMEM((CHUNK,), jnp.int32),
            pltpu.VMEM((CHUNK,), jnp.int32),
            pltpu.VMEM((CHUNK, HALF), jnp.float32),
            pltpu.VMEM((CHUNK, HALF), jnp.float32),
            pltpu.VMEM((CHUNK,), jnp.float32),
            pltpu.VMEM((256,), jnp.float32),
            pltpu.SemaphoreType.DMA,
            pltpu.SemaphoreType.DMA,
            pltpu.SemaphoreType.DMA,
            pltpu.SemaphoreType.DMA,
        ],
    )
    def kern(xs_hbm, xd_hbm, si_hbm, di_hbm, out_hbm,
             si_a, di_a, sr_a, dr_a, si_b, di_b, sr_b, dr_b,
             out_v, part_v, sem_a, sem_b, sem_ia, sem_ib):
        wid = lax.axis_index("s") * NUM_CORES + lax.axis_index("c")
        base0 = wid * EDGES_PER_W
        lane = lax.iota(jnp.int32, 16)

        def start_idx(j, si_v, di_v, sem_i):
            base = base0 + j * CHUNK
            pltpu.async_copy(si_hbm.at[pl.ds(base, CHUNK)], si_v, sem_i)
            pltpu.async_copy(di_hbm.at[pl.ds(base, CHUNK)], di_v, sem_i)

        def wait_idx(j, si_v, di_v, sem_i):
            base = base0 + j * CHUNK
            pltpu.make_async_copy(
                si_hbm.at[pl.ds(base, CHUNK)], si_v, sem_i).wait()
            pltpu.make_async_copy(
                di_hbm.at[pl.ds(base, CHUNK)], di_v, sem_i).wait()

        HC = CHUNK // 2

        def start_rows(si_v, di_v, srows, drows, sem):
            pltpu.async_copy(
                xs_hbm.at[si_v.at[pl.ds(0, HC)]], srows.at[pl.ds(0, HC)], sem)
            pltpu.async_copy(
                xs_hbm.at[si_v.at[pl.ds(HC, HC)]], srows.at[pl.ds(HC, HC)],
                sem)
            pltpu.async_copy(
                xd_hbm.at[di_v.at[pl.ds(0, HC)]], drows.at[pl.ds(0, HC)], sem)
            pltpu.async_copy(
                xd_hbm.at[di_v.at[pl.ds(HC, HC)]], drows.at[pl.ds(HC, HC)],
                sem)

        def drain(si_v, di_v, srows, drows, sem):
            for lo in (0, HC):
                pltpu.make_async_copy(
                    xs_hbm.at[si_v.at[pl.ds(lo, HC)]],
                    srows.at[pl.ds(lo, HC)], sem).wait()
                pltpu.make_async_copy(
                    xd_hbm.at[di_v.at[pl.ds(lo, HC)]],
                    drows.at[pl.ds(lo, HC)], sem).wait()

        def compute(j, srows, drows):
            def group_body(g, carry2):
                for e in range(16):
                    row = jnp.full((16,), g * 16 + e, jnp.int32)
                    partial = jnp.zeros((16,), jnp.float32)
                    for k in range(HALF // 16):
                        col = k * 16 + lane
                        sv = plsc.load_gather(srows, [row, col])
                        dv = plsc.load_gather(drows, [row, col])
                        partial = partial + sv * dv
                    part_v[pl.ds(e * 16, 16)] = partial
                acc = jnp.zeros((16,), jnp.float32)
                for d in range(16):
                    diag = lane * 16 + ((d + lane) & 15)
                    acc = acc + plsc.load_gather(part_v, [diag])
                out_v[pl.ds(g * 16, 16)] = acc
                return carry2

            lax.fori_loop(0, GROUPS, group_body, 0, unroll=False)
            base = base0 + j * CHUNK
            pltpu.sync_copy(out_v, out_hbm.at[pl.ds(base, CHUNK)])

        start_idx(0, si_a, di_a, sem_ia)
        wait_idx(0, si_a, di_a, sem_ia)
        start_rows(si_a, di_a, sr_a, dr_a, sem_a)
        start_idx(1, si_b, di_b, sem_ib)

        def pair_body(t, carry, prefetch_last=True):
            j0 = 2 * t
            wait_idx(j0 + 1, si_b, di_b, sem_ib)
            start_rows(si_b, di_b, sr_b, dr_b, sem_b)
            drain(si_a, di_a, sr_a, dr_a, sem_a)
            start_idx(j0 + 2, si_a, di_a, sem_ia)
            compute(j0, sr_a, dr_a)
            wait_idx(j0 + 2, si_a, di_a, sem_ia)
            start_rows(si_a, di_a, sr_a, dr_a, sem_a)
            drain(si_b, di_b, sr_b, dr_b, sem_b)
            if prefetch_last:
                start_idx(j0 + 3, si_b, di_b, sem_ib)
            compute(j0 + 1, sr_b, dr_b)
            return carry

        lax.fori_loop(0, NPAIRS - 1, pair_body, 0, unroll=False)
        pair_body(NPAIRS - 1, 0, prefetch_last=False)

        drain(si_a, di_a, sr_a, dr_a, sem_a)
        compute(NCHUNKS - 1, sr_a, dr_a)

    return kern


_KERNEL = _make_kernel()


@jax.jit
def kernel(x, edge_label_index):
    xs = x[:, :HALF]
    xd = x[:, HALF:]
    si = edge_label_index[0]
    di = edge_label_index[1]
    return _KERNEL(xs, xd, si, di)


# speedup vs baseline: 1.0038x; 1.0038x over previous
import functools

import jax
import jax.numpy as jnp
from jax import lax
from jax.experimental import pallas as pl
from jax.experimental.pallas import tpu as pltpu
from jax.experimental.pallas import tpu_sc as plsc

N_NODES = 10000
D_FEAT = 128
HALF = 64
N_EDGES = 320000

NUM_CORES = 2
NUM_SUBCORES = 16
NW = NUM_CORES * NUM_SUBCORES
EDGES_PER_W = N_EDGES // NW
CHUNK = 400
NCHUNKS = EDGES_PER_W // CHUNK
GROUPS = CHUNK // 16
NPAIRS = NCHUNKS // 2


def _make_kernel():
    mesh = plsc.VectorSubcoreMesh(core_axis_name="c", subcore_axis_name="s")

    @functools.partial(
        pl.kernel,
        mesh=mesh,
        compiler_params=pltpu.CompilerParams(
            needs_layout_passes=False, use_tc_tiling_on_sc=False),
        out_type=jax.ShapeDtypeStruct((N_EDGES,), jnp.float32),
        scratch_types=[
            pltpu.VMEM((CHUNK,), jnp.int32),
            pltpu.VMEM((CHUNK,), jnp.int32),
            pltpu.VMEM((CHUNK, HALF), jnp.float32),
            pltpu.VMEM((CHUNK, HALF), jnp.float32),
            pltpu.VMEM((CHUNK,), jnp.int32),
            pltpu.VMEM((CHUNK,), jnp.int32),
            pltpu.VMEM((CHUNK, HALF), jnp.float32),
            pltpu.VMEM((CHUNK, HALF), jnp.float32),
            pltpu.VMEM((CHUNK,), jnp.float32),
            pltpu.VMEM((CHUNK,), jnp.float32),
            pltpu.VMEM((256,), jnp.float32),
            pltpu.SemaphoreType.DMA,
            pltpu.SemaphoreType.DMA,
            pltpu.SemaphoreType.DMA,
            pltpu.SemaphoreType.DMA,
            pltpu.SemaphoreType.DMA,
            pltpu.SemaphoreType.DMA,
        ],
    )
    def kern(xs_hbm, xd_hbm, si_hbm, di_hbm, out_hbm,
             si_a, di_a, sr_a, dr_a, si_b, di_b, sr_b, dr_b,
             out_a, out_b, part_v, sem_a, sem_b, sem_ia, sem_ib,
             sem_oa, sem_ob):
        wid = lax.axis_index("s") * NUM_CORES + lax.axis_index("c")
        base0 = wid * EDGES_PER_W
        lane = lax.iota(jnp.int32, 16)

        def start_idx(j, si_v, di_v, sem_i):
            base = base0 + j * CHUNK
            pltpu.async_copy(si_hbm.at[pl.ds(base, CHUNK)], si_v, sem_i)
            pltpu.async_copy(di_hbm.at[pl.ds(base, CHUNK)], di_v, sem_i)

        def wait_idx(j, si_v, di_v, sem_i):
            base = base0 + j * CHUNK
            pltpu.make_async_copy(
                si_hbm.at[pl.ds(base, CHUNK)], si_v, sem_i).wait()
            pltpu.make_async_copy(
                di_hbm.at[pl.ds(base, CHUNK)], di_v, sem_i).wait()

        def start_rows(si_v, di_v, srows, drows, sem):
            pltpu.async_copy(xs_hbm.at[si_v], srows, sem)
            pltpu.async_copy(xd_hbm.at[di_v], drows, sem)

        def drain(si_v, di_v, srows, drows, sem):
            pltpu.make_async_copy(xs_hbm.at[si_v], srows, sem).wait()
            pltpu.make_async_copy(xd_hbm.at[di_v], drows, sem).wait()

        def wait_out(j, out_v, sem_o):
            base = base0 + j * CHUNK
            pltpu.make_async_copy(
                out_v, out_hbm.at[pl.ds(base, CHUNK)], sem_o).wait()

        def compute(j, srows, drows, out_v, sem_o, jm2=None):
            if jm2 is not None:
                wait_out(jm2, out_v, sem_o)

            def group_body(g, carry2):
                for e in range(16):
                    row = jnp.full((16,), g * 16 + e, jnp.int32)
                    partial = jnp.zeros((16,), jnp.float32)
                    for k in range(HALF // 16):
                        col = k * 16 + lane
                        sv = plsc.load_gather(srows, [row, col])
                        dv = plsc.load_gather(drows, [row, col])
                        partial = partial + sv * dv
                    part_v[pl.ds(e * 16, 16)] = partial
                acc = jnp.zeros((16,), jnp.float32)
                for d in range(16):
                    diag = lane * 16 + ((d + lane) & 15)
                    acc = acc + plsc.load_gather(part_v, [diag])
                out_v[pl.ds(g * 16, 16)] = acc
                return carry2

            lax.fori_loop(0, GROUPS, group_body, 0, unroll=False)
            base = base0 + j * CHUNK
            pltpu.async_copy(out_v, out_hbm.at[pl.ds(base, CHUNK)], sem_o)

        start_idx(0, si_a, di_a, sem_ia)
        wait_idx(0, si_a, di_a, sem_ia)
        start_rows(si_a, di_a, sr_a, dr_a, sem_a)
        start_idx(1, si_b, di_b, sem_ib)

        def pair_body(t, carry, prefetch_last=True, wait_prev=True):
            j0 = 2 * t
            wait_idx(j0 + 1, si_b, di_b, sem_ib)
            start_rows(si_b, di_b, sr_b, dr_b, sem_b)
            drain(si_a, di_a, sr_a, dr_a, sem_a)
            start_idx(j0 + 2, si_a, di_a, sem_ia)
            compute(j0, sr_a, dr_a, out_a, sem_oa,
                    j0 - 2 if wait_prev else None)
            wait_idx(j0 + 2, si_a, di_a, sem_ia)
            start_rows(si_a, di_a, sr_a, dr_a, sem_a)
            drain(si_b, di_b, sr_b, dr_b, sem_b)
            if prefetch_last:
                start_idx(j0 + 3, si_b, di_b, sem_ib)
            compute(j0 + 1, sr_b, dr_b, out_b, sem_ob,
                    j0 - 1 if wait_prev else None)
            return carry

        pair_body(0, 0, wait_prev=False)
        lax.fori_loop(1, NPAIRS - 1, pair_body, 0, unroll=False)
        pair_body(NPAIRS - 1, 0, prefetch_last=False)

        drain(si_a, di_a, sr_a, dr_a, sem_a)
        compute(NCHUNKS - 1, sr_a, dr_a, out_a, sem_oa, NCHUNKS - 3)
        wait_out(NCHUNKS - 2, out_b, sem_ob)
        wait_out(NCHUNKS - 1, out_a, sem_oa)

    return kern


_KERNEL = _make_kernel()


@jax.jit
def kernel(x, edge_label_index):
    xs = x[:, :HALF]
    xd = x[:, HALF:]
    si = edge_label_index[0]
    di = edge_label_index[1]
    return _KERNEL(xs, xd, si, di)


# speedup vs baseline: 1.2968x; 1.2918x over previous
import functools

import jax
import jax.numpy as jnp
from jax import lax
from jax.experimental import pallas as pl
from jax.experimental.pallas import tpu as pltpu
from jax.experimental.pallas import tpu_sc as plsc

N_NODES = 10000
D_FEAT = 128
HALF = 64
N_EDGES = 320000

NUM_CORES = 2
NUM_SUBCORES = 16
NW = NUM_CORES * NUM_SUBCORES
EDGES_PER_W = N_EDGES // NW
CHUNK = 400
NCHUNKS = EDGES_PER_W // CHUNK
GROUPS = CHUNK // 16
NPAIRS = NCHUNKS // 2


def _make_kernel():
    mesh = plsc.VectorSubcoreMesh(core_axis_name="c", subcore_axis_name="s")

    @functools.partial(
        pl.kernel,
        mesh=mesh,
        compiler_params=pltpu.CompilerParams(
            needs_layout_passes=False, use_tc_tiling_on_sc=False),
        out_type=jax.ShapeDtypeStruct((N_EDGES,), jnp.float32),
        scratch_types=[
            pltpu.VMEM((CHUNK,), jnp.int32),
            pltpu.VMEM((CHUNK,), jnp.int32),
            pltpu.VMEM((CHUNK, HALF), jnp.float32),
            pltpu.VMEM((CHUNK, HALF), jnp.float32),
            pltpu.VMEM((CHUNK,), jnp.int32),
            pltpu.VMEM((CHUNK,), jnp.int32),
            pltpu.VMEM((CHUNK, HALF), jnp.float32),
            pltpu.VMEM((CHUNK, HALF), jnp.float32),
            pltpu.VMEM((CHUNK,), jnp.float32),
            pltpu.VMEM((CHUNK,), jnp.float32),
            pltpu.VMEM((256,), jnp.float32),
            pltpu.SemaphoreType.DMA,
            pltpu.SemaphoreType.DMA,
            pltpu.SemaphoreType.DMA,
            pltpu.SemaphoreType.DMA,
            pltpu.SemaphoreType.DMA,
            pltpu.SemaphoreType.DMA,
        ],
    )
    def kern(xs_hbm, xd_hbm, si_hbm, di_hbm, out_hbm,
             si_a, di_a, sr_a, dr_a, si_b, di_b, sr_b, dr_b,
             out_a, out_b, part_v, sem_a, sem_b, sem_ia, sem_ib,
             sem_oa, sem_ob):
        wid = lax.axis_index("s") * NUM_CORES + lax.axis_index("c")
        base0 = wid * EDGES_PER_W
        lane = lax.iota(jnp.int32, 16)

        def start_idx(j, si_v, di_v, sem_i):
            base = base0 + j * CHUNK
            pltpu.async_copy(si_hbm.at[pl.ds(base, CHUNK)], si_v, sem_i)
            pltpu.async_copy(di_hbm.at[pl.ds(base, CHUNK)], di_v, sem_i)

        def wait_idx(j, si_v, di_v, sem_i):
            base = base0 + j * CHUNK
            pltpu.make_async_copy(
                si_hbm.at[pl.ds(base, CHUNK)], si_v, sem_i).wait()
            pltpu.make_async_copy(
                di_hbm.at[pl.ds(base, CHUNK)], di_v, sem_i).wait()

        def start_rows(si_v, di_v, srows, drows, sem):
            pltpu.async_copy(xs_hbm.at[si_v], srows, sem)
            pltpu.async_copy(xd_hbm.at[di_v], drows, sem)

        def drain(si_v, di_v, srows, drows, sem):
            pltpu.make_async_copy(xs_hbm.at[si_v], srows, sem).wait()
            pltpu.make_async_copy(xd_hbm.at[di_v], drows, sem).wait()

        def wait_out(j, out_v, sem_o):
            base = base0 + j * CHUNK
            pltpu.make_async_copy(
                out_v, out_hbm.at[pl.ds(base, CHUNK)], sem_o).wait()

        def compute(j, srows, drows, out_v, sem_o, jm2=None):
            if jm2 is not None:
                wait_out(jm2, out_v, sem_o)

            def rot(v, s):
                return v.at[(lane + s) & 15].get(mode="promise_in_bounds")

            def group_body(g, carry2):
                vs = []
                for e in range(16):
                    row = jnp.full((16,), g * 16 + e, jnp.int32)
                    partial = jnp.zeros((16,), jnp.float32)
                    for k in range(HALF // 16):
                        col = k * 16 + lane
                        sv = plsc.load_gather(srows, [row, col])
                        dv = plsc.load_gather(drows, [row, col])
                        partial = partial + sv * dv
                    vs.append(partial)
                for k in (1, 2, 4, 8):
                    mask = (lane & k) == 0
                    vs = [
                        jnp.where(mask, x, rot(y, -k))
                        + jnp.where(mask, rot(x, k), y)
                        for x, y in zip(vs[0::2], vs[1::2])
                    ]
                out_v[pl.ds(g * 16, 16)] = vs[0]
                return carry2

            lax.fori_loop(0, GROUPS, group_body, 0, unroll=False)
            base = base0 + j * CHUNK
            pltpu.async_copy(out_v, out_hbm.at[pl.ds(base, CHUNK)], sem_o)

        start_idx(0, si_a, di_a, sem_ia)
        wait_idx(0, si_a, di_a, sem_ia)
        start_rows(si_a, di_a, sr_a, dr_a, sem_a)
        start_idx(1, si_b, di_b, sem_ib)

        def pair_body(t, carry, prefetch_last=True, wait_prev=True):
            j0 = 2 * t
            wait_idx(j0 + 1, si_b, di_b, sem_ib)
            start_rows(si_b, di_b, sr_b, dr_b, sem_b)
            drain(si_a, di_a, sr_a, dr_a, sem_a)
            start_idx(j0 + 2, si_a, di_a, sem_ia)
            compute(j0, sr_a, dr_a, out_a, sem_oa,
                    j0 - 2 if wait_prev else None)
            wait_idx(j0 + 2, si_a, di_a, sem_ia)
            start_rows(si_a, di_a, sr_a, dr_a, sem_a)
            drain(si_b, di_b, sr_b, dr_b, sem_b)
            if prefetch_last:
                start_idx(j0 + 3, si_b, di_b, sem_ib)
            compute(j0 + 1, sr_b, dr_b, out_b, sem_ob,
                    j0 - 1 if wait_prev else None)
            return carry

        pair_body(0, 0, wait_prev=False)
        lax.fori_loop(1, NPAIRS - 1, pair_body, 0, unroll=False)
        pair_body(NPAIRS - 1, 0, prefetch_last=False)

        drain(si_a, di_a, sr_a, dr_a, sem_a)
        compute(NCHUNKS - 1, sr_a, dr_a, out_a, sem_oa, NCHUNKS - 3)
        wait_out(NCHUNKS - 2, out_b, sem_ob)
        wait_out(NCHUNKS - 1, out_a, sem_oa)

    return kern


_KERNEL = _make_kernel()


@jax.jit
def kernel(x, edge_label_index):
    xs = x[:, :HALF]
    xd = x[:, HALF:]
    si = edge_label_index[0]
    di = edge_label_index[1]
    return _KERNEL(xs, xd, si, di)


# speedup vs baseline: 1.3959x; 1.0764x over previous
import functools

import jax
import jax.numpy as jnp
from jax import lax
from jax.experimental import pallas as pl
from jax.experimental.pallas import tpu as pltpu
from jax.experimental.pallas import tpu_sc as plsc

N_NODES = 10000
D_FEAT = 128
HALF = 64
N_EDGES = 320000

NUM_CORES = 2
NUM_SUBCORES = 16
NW = NUM_CORES * NUM_SUBCORES
EDGES_PER_W = N_EDGES // NW
CHUNK = 400
NCHUNKS = EDGES_PER_W // CHUNK
GROUPS = CHUNK // 16
NPAIRS = NCHUNKS // 2
WORDS = HALF // 2


def _make_kernel():
    mesh = plsc.VectorSubcoreMesh(core_axis_name="c", subcore_axis_name="s")

    @functools.partial(
        pl.kernel,
        mesh=mesh,
        compiler_params=pltpu.CompilerParams(
            needs_layout_passes=False, use_tc_tiling_on_sc=False),
        out_type=jax.ShapeDtypeStruct((N_EDGES,), jnp.float32),
        scratch_types=[
            pltpu.VMEM((CHUNK,), jnp.int32),
            pltpu.VMEM((CHUNK,), jnp.int32),
            pltpu.VMEM((CHUNK, WORDS), jnp.int32),
            pltpu.VMEM((CHUNK, WORDS), jnp.int32),
            pltpu.VMEM((CHUNK,), jnp.int32),
            pltpu.VMEM((CHUNK,), jnp.int32),
            pltpu.VMEM((CHUNK, WORDS), jnp.int32),
            pltpu.VMEM((CHUNK, WORDS), jnp.int32),
            pltpu.VMEM((CHUNK,), jnp.float32),
            pltpu.VMEM((CHUNK,), jnp.float32),
            pltpu.VMEM((256,), jnp.float32),
            pltpu.SemaphoreType.DMA,
            pltpu.SemaphoreType.DMA,
            pltpu.SemaphoreType.DMA,
            pltpu.SemaphoreType.DMA,
            pltpu.SemaphoreType.DMA,
            pltpu.SemaphoreType.DMA,
        ],
    )
    def kern(xs_hbm, xd_hbm, si_hbm, di_hbm, out_hbm,
             si_a, di_a, sr_a, dr_a, si_b, di_b, sr_b, dr_b,
             out_a, out_b, part_v, sem_a, sem_b, sem_ia, sem_ib,
             sem_oa, sem_ob):
        wid = lax.axis_index("s") * NUM_CORES + lax.axis_index("c")
        base0 = wid * EDGES_PER_W
        lane = lax.iota(jnp.int32, 16)

        def start_idx(j, si_v, di_v, sem_i):
            base = base0 + j * CHUNK
            pltpu.async_copy(si_hbm.at[pl.ds(base, CHUNK)], si_v, sem_i)
            pltpu.async_copy(di_hbm.at[pl.ds(base, CHUNK)], di_v, sem_i)

        def wait_idx(j, si_v, di_v, sem_i):
            base = base0 + j * CHUNK
            pltpu.make_async_copy(
                si_hbm.at[pl.ds(base, CHUNK)], si_v, sem_i).wait()
            pltpu.make_async_copy(
                di_hbm.at[pl.ds(base, CHUNK)], di_v, sem_i).wait()

        def start_rows(si_v, di_v, srows, drows, sem):
            pltpu.async_copy(xs_hbm.at[si_v], srows, sem)
            pltpu.async_copy(xd_hbm.at[di_v], drows, sem)

        def drain(si_v, di_v, srows, drows, sem):
            pltpu.make_async_copy(xs_hbm.at[si_v], srows, sem).wait()
            pltpu.make_async_copy(xd_hbm.at[di_v], drows, sem).wait()

        def wait_out(j, out_v, sem_o):
            base = base0 + j * CHUNK
            pltpu.make_async_copy(
                out_v, out_hbm.at[pl.ds(base, CHUNK)], sem_o).wait()

        def compute(j, srows, drows, out_v, sem_o, jm2=None):
            if jm2 is not None:
                wait_out(jm2, out_v, sem_o)

            def rot(v, s):
                return v.at[(lane + s) & 15].get(mode="promise_in_bounds")

            def group_body(g, carry2):
                vs = []
                for e in range(16):
                    row = jnp.full((16,), g * 16 + e, jnp.int32)
                    partial = jnp.zeros((16,), jnp.float32)
                    for k in range(WORDS // 16):
                        col = k * 16 + lane
                        sw = plsc.load_gather(srows, [row, col])
                        dw = plsc.load_gather(drows, [row, col])
                        s0, s1 = plsc.unpack(
                            plsc.bitcast(sw, jnp.bfloat16),
                            format=plsc.PackFormat.INTERLEAVED)
                        d0, d1 = plsc.unpack(
                            plsc.bitcast(dw, jnp.bfloat16),
                            format=plsc.PackFormat.INTERLEAVED)
                        partial = partial + s0 * d0 + s1 * d1
                    vs.append(partial)
                for k in (1, 2, 4, 8):
                    mask = (lane & k) == 0
                    vs = [
                        jnp.where(mask, x, rot(y, -k))
                        + jnp.where(mask, rot(x, k), y)
                        for x, y in zip(vs[0::2], vs[1::2])
                    ]
                out_v[pl.ds(g * 16, 16)] = vs[0]
                return carry2

            lax.fori_loop(0, GROUPS, group_body, 0, unroll=False)
            base = base0 + j * CHUNK
            pltpu.async_copy(out_v, out_hbm.at[pl.ds(base, CHUNK)], sem_o)

        start_idx(0, si_a, di_a, sem_ia)
        wait_idx(0, si_a, di_a, sem_ia)
        start_rows(si_a, di_a, sr_a, dr_a, sem_a)
        start_idx(1, si_b, di_b, sem_ib)

        def pair_body(t, carry, prefetch_last=True, wait_prev=True):
            j0 = 2 * t
            wait_idx(j0 + 1, si_b, di_b, sem_ib)
            start_rows(si_b, di_b, sr_b, dr_b, sem_b)
            drain(si_a, di_a, sr_a, dr_a, sem_a)
            start_idx(j0 + 2, si_a, di_a, sem_ia)
            compute(j0, sr_a, dr_a, out_a, sem_oa,
                    j0 - 2 if wait_prev else None)
            wait_idx(j0 + 2, si_a, di_a, sem_ia)
            start_rows(si_a, di_a, sr_a, dr_a, sem_a)
            drain(si_b, di_b, sr_b, dr_b, sem_b)
            if prefetch_last:
                start_idx(j0 + 3, si_b, di_b, sem_ib)
            compute(j0 + 1, sr_b, dr_b, out_b, sem_ob,
                    j0 - 1 if wait_prev else None)
            return carry

        pair_body(0, 0, wait_prev=False)
        lax.fori_loop(1, NPAIRS - 1, pair_body, 0, unroll=False)
        pair_body(NPAIRS - 1, 0, prefetch_last=False)

        drain(si_a, di_a, sr_a, dr_a, sem_a)
        compute(NCHUNKS - 1, sr_a, dr_a, out_a, sem_oa, NCHUNKS - 3)
        wait_out(NCHUNKS - 2, out_b, sem_ob)
        wait_out(NCHUNKS - 1, out_a, sem_oa)

    return kern


_KERNEL = _make_kernel()


@jax.jit
def kernel(x, edge_label_index):
    xb = x.astype(jnp.bfloat16).reshape(N_NODES, HALF, 2)
    xs = jax.lax.bitcast_convert_type(xb[:, : HALF // 2], jnp.int32)
    xd = jax.lax.bitcast_convert_type(xb[:, HALF // 2:], jnp.int32)
    si = edge_label_index[0]
    di = edge_label_index[1]
    return _KERNEL(xs, xd, si, di)


# speedup vs baseline: 1.4737x; 1.0558x over previous
import functools

import jax
import jax.numpy as jnp
from jax import lax
from jax.experimental import pallas as pl
from jax.experimental.pallas import tpu as pltpu
from jax.experimental.pallas import tpu_sc as plsc

N_NODES = 10000
D_FEAT = 128
HALF = 64
N_EDGES = 320000

NUM_CORES = 2
NUM_SUBCORES = 16
NW = NUM_CORES * NUM_SUBCORES
EDGES_PER_W = N_EDGES // NW
CHUNK = 400
NCHUNKS = EDGES_PER_W // CHUNK
GROUPS = CHUNK // 16
NPAIRS = NCHUNKS // 2
WORDS = HALF // 2


def _make_kernel():
    mesh = plsc.VectorSubcoreMesh(core_axis_name="c", subcore_axis_name="s")

    @functools.partial(
        pl.kernel,
        mesh=mesh,
        compiler_params=pltpu.CompilerParams(
            needs_layout_passes=False, use_tc_tiling_on_sc=False),
        out_type=jax.ShapeDtypeStruct((N_EDGES,), jnp.float32),
        scratch_types=[
            pltpu.VMEM((CHUNK,), jnp.int32),
            pltpu.VMEM((CHUNK,), jnp.int32),
            pltpu.VMEM((CHUNK, WORDS), jnp.int32),
            pltpu.VMEM((CHUNK, WORDS), jnp.int32),
            pltpu.VMEM((CHUNK,), jnp.int32),
            pltpu.VMEM((CHUNK,), jnp.int32),
            pltpu.VMEM((CHUNK, WORDS), jnp.int32),
            pltpu.VMEM((CHUNK, WORDS), jnp.int32),
            pltpu.VMEM((CHUNK,), jnp.float32),
            pltpu.VMEM((CHUNK,), jnp.float32),
            pltpu.VMEM((256,), jnp.float32),
            pltpu.SemaphoreType.DMA,
            pltpu.SemaphoreType.DMA,
            pltpu.SemaphoreType.DMA,
            pltpu.SemaphoreType.DMA,
            pltpu.SemaphoreType.DMA,
            pltpu.SemaphoreType.DMA,
        ],
    )
    def kern(xs_hbm, xd_hbm, si_hbm, di_hbm, out_hbm,
             si_a, di_a, sr_a, dr_a, si_b, di_b, sr_b, dr_b,
             out_a, out_b, part_v, sem_a, sem_b, sem_ia, sem_ib,
             sem_oa, sem_ob):
        wid = lax.axis_index("s") * NUM_CORES + lax.axis_index("c")
        base0 = wid * EDGES_PER_W
        lane = lax.iota(jnp.int32, 16)

        def start_idx(j, si_v, di_v, sem_i):
            base = base0 + j * CHUNK
            pltpu.async_copy(si_hbm.at[pl.ds(base, CHUNK)], si_v, sem_i)
            pltpu.async_copy(di_hbm.at[pl.ds(base, CHUNK)], di_v, sem_i)

        def wait_idx(j, si_v, di_v, sem_i):
            base = base0 + j * CHUNK
            pltpu.make_async_copy(
                si_hbm.at[pl.ds(base, CHUNK)], si_v, sem_i).wait()
            pltpu.make_async_copy(
                di_hbm.at[pl.ds(base, CHUNK)], di_v, sem_i).wait()

        def start_rows(si_v, di_v, srows, drows, sem):
            pltpu.async_copy(xs_hbm.at[si_v], srows, sem)
            pltpu.async_copy(xd_hbm.at[di_v], drows, sem)

        def drain(si_v, di_v, srows, drows, sem):
            pltpu.make_async_copy(xs_hbm.at[si_v], srows, sem).wait()
            pltpu.make_async_copy(xd_hbm.at[di_v], drows, sem).wait()

        def wait_out(j, out_v, sem_o):
            base = base0 + j * CHUNK
            pltpu.make_async_copy(
                out_v, out_hbm.at[pl.ds(base, CHUNK)], sem_o).wait()

        def compute(j, srows, drows, out_v, sem_o, jm2=None):
            if jm2 is not None:
                wait_out(jm2, out_v, sem_o)

            def rot(v, s):
                return v.at[(lane + s) & 15].get(mode="promise_in_bounds")

            def group_body(g, carry2):
                vs = []
                for e in range(16):
                    row = jnp.full((16,), g * 16 + e, jnp.int32)
                    partial = jnp.zeros((16,), jnp.float32)
                    for k in range(WORDS // 16):
                        col = k * 16 + lane
                        sw = plsc.load_gather(srows, [row, col])
                        dw = plsc.load_gather(drows, [row, col])
                        pb = (plsc.bitcast(sw, jnp.bfloat16)
                              * plsc.bitcast(dw, jnp.bfloat16))
                        p0, p1 = plsc.unpack(
                            pb, format=plsc.PackFormat.INTERLEAVED)
                        partial = partial + p0 + p1
                    vs.append(partial)
                for k in (1, 2, 4, 8):
                    mask = (lane & k) == 0
                    vs = [
                        jnp.where(mask, x, rot(y, -k))
                        + jnp.where(mask, rot(x, k), y)
                        for x, y in zip(vs[0::2], vs[1::2])
                    ]
                out_v[pl.ds(g * 16, 16)] = vs[0]
                return carry2

            lax.fori_loop(0, GROUPS, group_body, 0, unroll=False)
            base = base0 + j * CHUNK
            pltpu.async_copy(out_v, out_hbm.at[pl.ds(base, CHUNK)], sem_o)

        start_idx(0, si_a, di_a, sem_ia)
        wait_idx(0, si_a, di_a, sem_ia)
        start_rows(si_a, di_a, sr_a, dr_a, sem_a)
        start_idx(1, si_b, di_b, sem_ib)

        def pair_body(t, carry, prefetch_last=True, wait_prev=True):
            j0 = 2 * t
            wait_idx(j0 + 1, si_b, di_b, sem_ib)
            start_rows(si_b, di_b, sr_b, dr_b, sem_b)
            drain(si_a, di_a, sr_a, dr_a, sem_a)
            start_idx(j0 + 2, si_a, di_a, sem_ia)
            compute(j0, sr_a, dr_a, out_a, sem_oa,
                    j0 - 2 if wait_prev else None)
            wait_idx(j0 + 2, si_a, di_a, sem_ia)
            start_rows(si_a, di_a, sr_a, dr_a, sem_a)
            drain(si_b, di_b, sr_b, dr_b, sem_b)
            if prefetch_last:
                start_idx(j0 + 3, si_b, di_b, sem_ib)
            compute(j0 + 1, sr_b, dr_b, out_b, sem_ob,
                    j0 - 1 if wait_prev else None)
            return carry

        pair_body(0, 0, wait_prev=False)
        lax.fori_loop(1, NPAIRS - 1, pair_body, 0, unroll=False)
        pair_body(NPAIRS - 1, 0, prefetch_last=False)

        drain(si_a, di_a, sr_a, dr_a, sem_a)
        compute(NCHUNKS - 1, sr_a, dr_a, out_a, sem_oa, NCHUNKS - 3)
        wait_out(NCHUNKS - 2, out_b, sem_ob)
        wait_out(NCHUNKS - 1, out_a, sem_oa)

    return kern


_KERNEL = _make_kernel()


@jax.jit
def kernel(x, edge_label_index):
    xb = x.astype(jnp.bfloat16).reshape(N_NODES, HALF, 2)
    xs = jax.lax.bitcast_convert_type(xb[:, : HALF // 2], jnp.int32)
    xd = jax.lax.bitcast_convert_type(xb[:, HALF // 2:], jnp.int32)
    si = edge_label_index[0]
    di = edge_label_index[1]
    return _KERNEL(xs, xd, si, di)
